# baseline (speedup 1.0000x reference)
import functools

import jax
import jax.numpy as jnp
from jax import lax
from jax.experimental import pallas as pl
from jax.experimental.pallas import tpu as pltpu
from jax.experimental.pallas import tpu_sc as plsc

N_NODES = 10000
N_EDGES = 320000
D = 128
H = D // 2
N_LAYERS = 3
NS = 16
EPT = N_EDGES // NS
K = 80
NCHUNK = EPT // K
REC = 2 * K
RPT = 640
N_PAD = NS * RPT
NP = N_PAD // 2
RPP = RPT // 2
SUB = RPP // K


def _idx(eb, s2, d2):
  for g in range(K // 16):
    sl = pl.ds(g * 16, 16)
    s2[sl] = lax.shift_right_logical(eb[sl], 1)
    d2[sl] = lax.shift_right_logical(eb[pl.ds(K + g * 16, 16)], 1)


def _scale(eb, wb, rows, outb):
  def group(g, _):
    w16 = wb[pl.ds(g * 16, 16)]
    ps16 = (eb[pl.ds(g * 16, 16)] & 1) * 64
    pd16 = (eb[pl.ds(K + g * 16, 16)] & 1) * 64
    z = jnp.zeros((16,), jnp.float32)
    for e16 in range(0, 16, 4):
      es = [g * 16 + e16 + u for u in range(4)]
      ws = [w16[e16 + u] for u in range(4)]
      pos = [ps16[e16 + u] for u in range(4)]
      qos = [pd16[e16 + u] for u in range(4)]
      qzs = [64 - q for q in qos]
      vs = [[rows[es[u], pl.ds(pos[u] + j * 16, 16)] for j in range(4)]
            for u in range(4)]
      for u in range(4):
        for j in range(4):
          outb[es[u], pl.ds(qos[u] + j * 16, 16)] = vs[u][j] * ws[u]
        for j in range(4):
          outb[es[u], pl.ds(qzs[u] + j * 16, 16)] = z
    return 0

  lax.fori_loop(0, K // 16, group, 0)


def _sc_body(emb2, edges, wts, out2, xs, acc, rows0, rows1, outb0, outb1,
             eb0, eb1, wb0, wb1, s20, s21, d20, d21, d2s0, d2s1,
             es0, es1, gs0, gs1, ss0, ss1):
  c = lax.axis_index("c")
  s = lax.axis_index("s")
  row0 = s * RPP
  cbase = s * NCHUNK

  def zero_rows(r, _):
    for j in range(8):
      rows0[r, pl.ds(j * 16, 16)] = jnp.zeros((16,), jnp.float32)
    return 0

  lax.fori_loop(0, K, zero_rows, 0)

  def init_chunk(t, _):
    r0 = row0 + t * K
    pltpu.sync_copy(rows0, acc.at[pl.ds(r0, K)])
    pltpu.sync_copy(emb2.at[c].at[pl.ds(r0, K)], rows1)
    pltpu.sync_copy(rows1, out2.at[c].at[pl.ds(r0, K)])
    return 0

  lax.fori_loop(0, SUB, init_chunk, 0)
  plsc.subcore_barrier()

  for l in range(N_LAYERS):
    last = l == N_LAYERS - 1
    xref = emb2.at[c] if l == 0 else xs.at[c]

    def do_chunk(i, eb_b, wb_b, rows_b, outb_b, s2_b, d2_b, d2s_b,
                 es_b, gs_b, ss_b,
                 eb_n, wb_n, rows_n, s2_n, d2_n, es_n, gs_n):
      @pl.when(i < NCHUNK - 1)
      def _():
        pltpu.make_async_copy(edges.at[cbase + i + 1], eb_n, es_n).wait()
        pltpu.make_async_copy(wts.at[cbase + i + 1], wb_n, es_n).wait()
        _idx(eb_n, s2_n, d2_n)
        pltpu.async_copy(xref.at[s2_n], rows_n, gs_n)

      pltpu.make_async_copy(xref.at[s2_b], rows_b, gs_b).wait()

      @pl.when(i >= 2)
      def _():
        pltpu.make_async_copy(outb_b, acc.at[d2s_b], ss_b).wait()

      for g in range(K // 16):
        sl = pl.ds(g * 16, 16)
        d2s_b[sl] = d2_b[sl]
      _scale(eb_b, wb_b, rows_b, outb_b)
      pltpu.async_copy(outb_b, acc.at[d2s_b], ss_b, add=True)

      @pl.when(i + 2 < NCHUNK)
      def _():
        pltpu.async_copy(edges.at[cbase + i + 2], eb_b, es_b)
        pltpu.async_copy(wts.at[cbase + i + 2], wb_b, es_b)

    pltpu.async_copy(edges.at[cbase], eb0, es0).wait()
    pltpu.async_copy(wts.at[cbase], wb0, es0).wait()
    _idx(eb0, s20, d20)
    pltpu.async_copy(xref.at[s20], rows0, gs0)
    pltpu.async_copy(edges.at[cbase + 1], eb1, es1)
    pltpu.async_copy(wts.at[cbase + 1], wb1, es1)

    def pair(t, _):
      i = 2 * t
      do_chunk(i, eb0, wb0, rows0, outb0, s20, d20, d2s0, es0, gs0, ss0,
               eb1, wb1, rows1, s21, d21, es1, gs1)
      do_chunk(i + 1, eb1, wb1, rows1, outb1, s21, d21, d2s1, es1, gs1, ss1,
               eb0, wb0, rows0, s20, d20, es0, gs0)
      return 0

    lax.fori_loop(0, NCHUNK // 2, pair, 0)
    pltpu.make_async_copy(outb0, acc.at[d2s0], ss0).wait()
    pltpu.make_async_copy(outb1, acc.at[d2s1], ss1).wait()
    plsc.subcore_barrier()

    def upd_chunk(t, _):
      r0 = row0 + t * K
      pltpu.sync_copy(acc.at[pl.ds(r0, K)], rows0)
      pltpu.sync_copy(out2.at[c].at[pl.ds(r0, K)], rows1)

      def fold(r, _):
        for j in range(8):
          sl = pl.ds(j * 16, 16)
          v = rows1[r, sl] + rows0[r, sl]
          rows1[r, sl] = v * 0.25 if last else v
        return 0

      lax.fori_loop(0, K, fold, 0)
      pltpu.sync_copy(rows1, out2.at[c].at[pl.ds(r0, K)])
      if not last:
        pltpu.sync_copy(rows0, xs.at[c].at[pl.ds(r0, K)])
        lax.fori_loop(0, K, zero_rows, 0)
        pltpu.sync_copy(rows0, acc.at[pl.ds(r0, K)])
      return 0

    lax.fori_loop(0, SUB, upd_chunk, 0)
    if not last:
      plsc.subcore_barrier()


_sc_call = functools.partial(
    pl.kernel,
    out_type=(jax.ShapeDtypeStruct((2, NP, D), jnp.float32),
              jax.ShapeDtypeStruct((2, NP, D), jnp.float32)),
    mesh=plsc.VectorSubcoreMesh(core_axis_name="c", subcore_axis_name="s"),
    scratch_types=[
        pltpu.VMEM_SHARED((NP, D), jnp.float32),
        pltpu.VMEM((K, D), jnp.float32),
        pltpu.VMEM((K, D), jnp.float32),
        pltpu.VMEM((K, D), jnp.float32),
        pltpu.VMEM((K, D), jnp.float32),
        pltpu.VMEM((REC,), jnp.int32),
        pltpu.VMEM((REC,), jnp.int32),
        pltpu.VMEM((K,), jnp.float32),
        pltpu.VMEM((K,), jnp.float32),
        pltpu.VMEM((K,), jnp.int32),
        pltpu.VMEM((K,), jnp.int32),
        pltpu.VMEM((K,), jnp.int32),
        pltpu.VMEM((K,), jnp.int32),
        pltpu.VMEM((K,), jnp.int32),
        pltpu.VMEM((K,), jnp.int32),
        pltpu.SemaphoreType.DMA,
        pltpu.SemaphoreType.DMA,
        pltpu.SemaphoreType.DMA,
        pltpu.SemaphoreType.DMA,
        pltpu.SemaphoreType.DMA,
        pltpu.SemaphoreType.DMA,
    ],
)(_sc_body)


def kernel(embedding, edge_index, edge_weight):
  emb2 = jnp.stack([embedding[:, :H], embedding[:, H:]])
  emb2 = jnp.pad(emb2, ((0, 0), (0, N_PAD - N_NODES), (0, 0)))
  emb2 = emb2.reshape(2, NP, D)

  src = edge_index[0].reshape(NS, NCHUNK, K)
  dst = edge_index[1].reshape(NS, NCHUNK, K)
  edges = jnp.concatenate([src, dst], axis=-1).reshape(NS * NCHUNK, REC)
  wts = edge_weight.reshape(NS * NCHUNK, K)

  out2, _ = _sc_call(emb2, edges, wts)
  halves = out2.reshape(2, N_PAD, H)[:, :N_NODES]
  return jnp.concatenate([halves[0], halves[1]], axis=1)


# speedup vs baseline: 1.0108x; 1.0108x over previous
import functools

import jax
import jax.numpy as jnp
from jax import lax
from jax.experimental import pallas as pl
from jax.experimental.pallas import tpu as pltpu
from jax.experimental.pallas import tpu_sc as plsc

N_NODES = 10000
N_EDGES = 320000
D = 128
H = D // 2
N_LAYERS = 3
NS = 16
EPT = N_EDGES // NS
K = 80
NCHUNK = EPT // K
REC = 2 * K
RPT = 640
N_PAD = NS * RPT
NP = N_PAD // 2
RPP = RPT // 2
SUB = RPP // K


def _idx(eb, s2, d2):
  for g in range(K // 16):
    sl = pl.ds(g * 16, 16)
    s2[sl] = lax.shift_right_logical(eb[sl], 1)
    d2[sl] = lax.shift_right_logical(eb[pl.ds(K + g * 16, 16)], 1)


def _scale(eb, wb, rows, outb):
  def group(g, _):
    w16 = wb[pl.ds(g * 16, 16)]
    ps16 = (eb[pl.ds(g * 16, 16)] & 1) * 64
    pdm = (eb[pl.ds(K + g * 16, 16)] & 1) == 0
    wlo16 = jnp.where(pdm, w16, 0.0)
    whi16 = w16 - wlo16
    for e16 in range(0, 16, 2):
      es = [g * 16 + e16 + u for u in range(2)]
      wlos = [wlo16[e16 + u] for u in range(2)]
      whis = [whi16[e16 + u] for u in range(2)]
      pos = [ps16[e16 + u] for u in range(2)]
      vs = [[rows[es[u], pl.ds(pos[u] + j * 16, 16)] for j in range(4)]
            for u in range(2)]
      for u in range(2):
        for j in range(4):
          outb[es[u], pl.ds(j * 16, 16)] = vs[u][j] * wlos[u]
        for j in range(4):
          outb[es[u], pl.ds(64 + j * 16, 16)] = vs[u][j] * whis[u]
    return 0

  lax.fori_loop(0, K // 16, group, 0)


def _sc_body(emb2, edges, wts, out2, xs, acc, rows0, rows1, outb0, outb1,
             eb0, eb1, wb0, wb1, s20, s21, d20, d21, d2s0, d2s1,
             es0, es1, gs0, gs1, ss0, ss1):
  c = lax.axis_index("c")
  s = lax.axis_index("s")
  row0 = s * RPP
  cbase = s * NCHUNK

  def zero_rows(r, _):
    for j in range(8):
      rows0[r, pl.ds(j * 16, 16)] = jnp.zeros((16,), jnp.float32)
    return 0

  lax.fori_loop(0, K, zero_rows, 0)

  def init_chunk(t, _):
    r0 = row0 + t * K
    pltpu.sync_copy(rows0, acc.at[pl.ds(r0, K)])
    pltpu.sync_copy(emb2.at[c].at[pl.ds(r0, K)], rows1)
    pltpu.sync_copy(rows1, out2.at[c].at[pl.ds(r0, K)])
    return 0

  lax.fori_loop(0, SUB, init_chunk, 0)
  plsc.subcore_barrier()

  for l in range(N_LAYERS):
    last = l == N_LAYERS - 1
    xref = emb2.at[c] if l == 0 else xs.at[c]

    def do_chunk(i, eb_b, wb_b, rows_b, outb_b, s2_b, d2_b, d2s_b,
                 es_b, gs_b, ss_b,
                 eb_n, wb_n, rows_n, s2_n, d2_n, es_n, gs_n):
      @pl.when(i < NCHUNK - 1)
      def _():
        pltpu.make_async_copy(edges.at[cbase + i + 1], eb_n, es_n).wait()
        pltpu.make_async_copy(wts.at[cbase + i + 1], wb_n, es_n).wait()
        _idx(eb_n, s2_n, d2_n)
        pltpu.async_copy(xref.at[s2_n], rows_n, gs_n)

      pltpu.make_async_copy(xref.at[s2_b], rows_b, gs_b).wait()

      @pl.when(i >= 2)
      def _():
        pltpu.make_async_copy(outb_b, acc.at[d2s_b], ss_b).wait()

      for g in range(K // 16):
        sl = pl.ds(g * 16, 16)
        d2s_b[sl] = d2_b[sl]
      _scale(eb_b, wb_b, rows_b, outb_b)
      pltpu.async_copy(outb_b, acc.at[d2s_b], ss_b, add=True)

      @pl.when(i + 2 < NCHUNK)
      def _():
        pltpu.async_copy(edges.at[cbase + i + 2], eb_b, es_b)
        pltpu.async_copy(wts.at[cbase + i + 2], wb_b, es_b)

    pltpu.async_copy(edges.at[cbase], eb0, es0).wait()
    pltpu.async_copy(wts.at[cbase], wb0, es0).wait()
    _idx(eb0, s20, d20)
    pltpu.async_copy(xref.at[s20], rows0, gs0)
    pltpu.async_copy(edges.at[cbase + 1], eb1, es1)
    pltpu.async_copy(wts.at[cbase + 1], wb1, es1)

    def pair(t, _):
      i = 2 * t
      do_chunk(i, eb0, wb0, rows0, outb0, s20, d20, d2s0, es0, gs0, ss0,
               eb1, wb1, rows1, s21, d21, es1, gs1)
      do_chunk(i + 1, eb1, wb1, rows1, outb1, s21, d21, d2s1, es1, gs1, ss1,
               eb0, wb0, rows0, s20, d20, es0, gs0)
      return 0

    lax.fori_loop(0, NCHUNK // 2, pair, 0)
    pltpu.make_async_copy(outb0, acc.at[d2s0], ss0).wait()
    pltpu.make_async_copy(outb1, acc.at[d2s1], ss1).wait()
    plsc.subcore_barrier()

    def upd_chunk(t, _):
      r0 = row0 + t * K
      pltpu.sync_copy(acc.at[pl.ds(r0, K)], rows0)
      pltpu.sync_copy(out2.at[c].at[pl.ds(r0, K)], rows1)

      def fold(r, _):
        for j in range(8):
          sl = pl.ds(j * 16, 16)
          v = rows1[r, sl] + rows0[r, sl]
          rows1[r, sl] = v * 0.25 if last else v
        return 0

      lax.fori_loop(0, K, fold, 0)
      pltpu.sync_copy(rows1, out2.at[c].at[pl.ds(r0, K)])
      if not last:
        pltpu.sync_copy(rows0, xs.at[c].at[pl.ds(r0, K)])
        lax.fori_loop(0, K, zero_rows, 0)
        pltpu.sync_copy(rows0, acc.at[pl.ds(r0, K)])
      return 0

    lax.fori_loop(0, SUB, upd_chunk, 0)
    if not last:
      plsc.subcore_barrier()


_sc_call = functools.partial(
    pl.kernel,
    out_type=(jax.ShapeDtypeStruct((2, NP, D), jnp.float32),
              jax.ShapeDtypeStruct((2, NP, D), jnp.float32)),
    mesh=plsc.VectorSubcoreMesh(core_axis_name="c", subcore_axis_name="s"),
    scratch_types=[
        pltpu.VMEM_SHARED((NP, D), jnp.float32),
        pltpu.VMEM((K, D), jnp.float32),
        pltpu.VMEM((K, D), jnp.float32),
        pltpu.VMEM((K, D), jnp.float32),
        pltpu.VMEM((K, D), jnp.float32),
        pltpu.VMEM((REC,), jnp.int32),
        pltpu.VMEM((REC,), jnp.int32),
        pltpu.VMEM((K,), jnp.float32),
        pltpu.VMEM((K,), jnp.float32),
        pltpu.VMEM((K,), jnp.int32),
        pltpu.VMEM((K,), jnp.int32),
        pltpu.VMEM((K,), jnp.int32),
        pltpu.VMEM((K,), jnp.int32),
        pltpu.VMEM((K,), jnp.int32),
        pltpu.VMEM((K,), jnp.int32),
        pltpu.SemaphoreType.DMA,
        pltpu.SemaphoreType.DMA,
        pltpu.SemaphoreType.DMA,
        pltpu.SemaphoreType.DMA,
        pltpu.SemaphoreType.DMA,
        pltpu.SemaphoreType.DMA,
    ],
)(_sc_body)


def kernel(embedding, edge_index, edge_weight):
  emb2 = jnp.stack([embedding[:, :H], embedding[:, H:]])
  emb2 = jnp.pad(emb2, ((0, 0), (0, N_PAD - N_NODES), (0, 0)))
  emb2 = emb2.reshape(2, NP, D)

  src = edge_index[0].reshape(NS, NCHUNK, K)
  dst = edge_index[1].reshape(NS, NCHUNK, K)
  edges = jnp.concatenate([src, dst], axis=-1).reshape(NS * NCHUNK, REC)
  wts = edge_weight.reshape(NS * NCHUNK, K)

  out2, _ = _sc_call(emb2, edges, wts)
  halves = out2.reshape(2, N_PAD, H)[:, :N_NODES]
  return jnp.concatenate([halves[0], halves[1]], axis=1)
